# baseline (speedup 1.0000x reference)
import functools

import jax
import jax.numpy as jnp
from jax import lax
from jax.experimental import pallas as pl
from jax.experimental.pallas import tpu as pltpu
from jax.experimental.pallas import tpu_sc as plsc

_IN_DIM = 101
_D = 64
_BATCH = 16384
_HIST = 200
_MAX_NORM = 1.0

_NC = 2
_NS = 16
_NW = _NC * _NS
_L = 16

_BT = _BATCH // 128
_CT = _D // 8
_BG = _BT // 8
_PAIRS = _HIST * _BG
_P_PW = _PAIRS // _NW

_XH_BYTES = 1024 * 4
_BLK_BYTES = 8 * 8 * 128 * 4


def _norm_body(tab_ref, out_ref):
    t = tab_ref[...]
    norms = jnp.sqrt(jnp.sum(t * t, axis=1, keepdims=True))
    scale = jnp.where(norms > _MAX_NORM, _MAX_NORM / (norms + 1e-7), 1.0)
    s = t * scale
    out_ref[...] = jnp.concatenate(
        [jnp.transpose(s), jnp.zeros((_D, 128 - _IN_DIM), jnp.float32)], axis=1
    )


def _normalize_table_t(table):
    return pl.pallas_call(
        _norm_body,
        out_shape=jax.ShapeDtypeStruct((_D, 128), jnp.float32),
    )(table)


def _gather_body(tab_hbm, xt_hbm, out_hbm, tab_v, xh0, xh1, blk0, blk1, sems):
    isem0, isem1, ssem0, ssem1 = sems
    pltpu.sync_copy(tab_hbm, tab_v)

    wid = lax.axis_index("s") * _NC + lax.axis_index("c")
    p_base = wid * _P_PW
    step128 = jnp.full((_L,), 128, jnp.int32)

    def xh_start(p, xh, isem):
        pp = lax.min(p_base + p, _PAIRS - 1)
        h = pp // _BG
        b0 = (pp % _BG) * 1024
        pltpu.async_copy(xt_hbm.at[h, pl.ds(b0, 1024)], xh, isem)

    def xh_wait(xh, isem):
        pltpu.make_async_copy(xt_hbm.at[0, pl.ds(0, 1024)], xh, isem).wait()

    def store_wait(blk, ssem):
        pltpu.make_async_copy(
            blk, out_hbm.at[0, 0, pl.ds(0, 8)], ssem
        ).wait()

    def gen_block(ct, xh, blk):
        base = jnp.full((_L,), ct * 1024, jnp.int32)

        @plsc.parallel_loop(0, 64, step=1, unroll=16)
        def _(g):
            bt = g // 8
            j16 = (g % 8) * _L
            xv = xh[pl.ds(g * _L, _L)]
            addr = xv + base
            for c8 in range(8):
                blk[bt, c8, pl.ds(j16, _L)] = plsc.load_gather(tab_v, [addr])
                if c8 != 7:
                    addr = addr + step128

    def phase(p, cp, ph, xh, blk, ssem):
        ct = cp * 2 + ph
        pp = p_base + p
        h = pp // _BG
        bt0 = (pp % _BG) * 8

        @pl.when(jnp.logical_or(p > 0, cp > 0))
        def _():
            store_wait(blk, ssem)

        gen_block(ct, xh, blk)
        pltpu.async_copy(blk, out_hbm.at[h, ct, pl.ds(bt0, 8)], ssem)

    def pair_work(p, xh):
        def cp_body(cp, carry):
            phase(p, cp, 0, xh, blk0, ssem0)
            phase(p, cp, 1, xh, blk1, ssem1)
            return carry

        lax.fori_loop(0, _CT // 2, cp_body, 0)

    xh_start(0, xh0, isem0)

    def p_body(p, carry):
        @pl.when(lax.rem(p, 2) == 0)
        def _():
            xh_wait(xh0, isem0)
            xh_start(p + 1, xh1, isem1)
            pair_work(p, xh0)

        @pl.when(lax.rem(p, 2) == 1)
        def _():
            xh_wait(xh1, isem1)
            xh_start(p + 1, xh0, isem0)
            pair_work(p, xh1)

        return carry

    lax.fori_loop(0, _P_PW, p_body, 0)

    store_wait(blk0, ssem0)
    store_wait(blk1, ssem1)
    xh_wait(xh0, isem0)


@functools.partial(
    pl.kernel,
    out_type=jax.ShapeDtypeStruct((_HIST, _CT, _BT, 8, 128), jnp.float32),
    mesh=plsc.VectorSubcoreMesh(core_axis_name="c", subcore_axis_name="s"),
    scratch_types=[
        pltpu.VMEM((_D * 128,), jnp.float32),
        pltpu.VMEM((1024,), jnp.int32),
        pltpu.VMEM((1024,), jnp.int32),
        pltpu.VMEM((8, 8, 128), jnp.float32),
        pltpu.VMEM((8, 8, 128), jnp.float32),
        pltpu.SemaphoreType.DMA,
        pltpu.SemaphoreType.DMA,
        pltpu.SemaphoreType.DMA,
        pltpu.SemaphoreType.DMA,
    ],
    compiler_params=pltpu.CompilerParams(
        use_tc_tiling_on_sc=False, needs_layout_passes=False
    ),
)
def _sc_gather(tab_hbm, xt_hbm, out_hbm, tab_v, xh0, xh1, blk0, blk1, *sems):
    _gather_body(tab_hbm, xt_hbm, out_hbm, tab_v, xh0, xh1, blk0, blk1, sems)


def kernel(x, table):
    tab_t = _normalize_table_t(table).reshape(_D * 128)
    out5 = _sc_gather(tab_t, x.T)
    return out5.transpose(2, 4, 0, 1, 3).reshape(_BATCH, _HIST, _D)


# speedup vs baseline: 1.1346x; 1.1346x over previous
import functools

import jax
import jax.numpy as jnp
from jax import lax
from jax.experimental import pallas as pl
from jax.experimental.pallas import tpu as pltpu
from jax.experimental.pallas import tpu_sc as plsc

_IN_DIM = 101
_D = 64
_BATCH = 16384
_HIST = 200
_MAX_NORM = 1.0

_NC = 2
_NS = 16
_NW = _NC * _NS
_L = 16

_BT = _BATCH // 128
_CT = _D // 8
_BG = _BT // 8
_PAIRS = _HIST * _BG
_P_PW = _PAIRS // _NW

_XH_BYTES = 1024 * 4
_BLK_BYTES = 8 * 8 * 128 * 4


def _norm_body(tab_ref, out_ref):
    t = tab_ref[...]
    norms = jnp.sqrt(jnp.sum(t * t, axis=1, keepdims=True))
    scale = jnp.where(norms > _MAX_NORM, _MAX_NORM / (norms + 1e-7), 1.0)
    s = t * scale
    out_ref[...] = jnp.concatenate(
        [jnp.transpose(s), jnp.zeros((_D, 128 - _IN_DIM), jnp.float32)], axis=1
    )


def _normalize_table_t(table):
    return pl.pallas_call(
        _norm_body,
        out_shape=jax.ShapeDtypeStruct((_D, 128), jnp.float32),
    )(table)


def _gather_body(tab_hbm, xt_hbm, out_hbm, tab_v, xh0, xh1, blk0, blk1, sems):
    isem0, isem1, ssem0, ssem1 = sems
    pltpu.sync_copy(tab_hbm, tab_v)

    wid = lax.axis_index("s") * _NC + lax.axis_index("c")
    p_base = wid * _P_PW
    step128 = jnp.full((_L,), 128, jnp.int32)

    def xh_start(p, xh, isem):
        pp = lax.min(p_base + p, _PAIRS - 1)
        h = pp // _BG
        b0 = (pp % _BG) * 1024
        pltpu.async_copy(xt_hbm.at[h, pl.ds(b0, 1024)], xh, isem)

    def xh_wait(xh, isem):
        pltpu.make_async_copy(xt_hbm.at[0, pl.ds(0, 1024)], xh, isem).wait()

    def store_wait(blk, ssem):
        pltpu.make_async_copy(
            blk, out_hbm.at[0, 0, pl.ds(0, 8)], ssem
        ).wait()

    def gen_block(ct, xh, blk):
        base = jnp.full((_L,), ct * 1024, jnp.int32)

        @plsc.parallel_loop(0, 64, step=1, unroll=8)
        def _(g):
            bt = g // 8
            j16 = (g % 8) * _L
            xv = xh[pl.ds(g * _L, _L)]
            addr = xv + base
            for c8 in range(8):
                blk[bt, c8, pl.ds(j16, _L)] = plsc.load_gather(tab_v, [addr])
                if c8 != 7:
                    addr = addr + step128

    def phase(p, cp, ph, xh, blk, ssem):
        ct = cp * 2 + ph
        pp = p_base + p
        h = pp // _BG
        bt0 = (pp % _BG) * 8

        @pl.when(jnp.logical_or(p > 0, cp > 0))
        def _():
            store_wait(blk, ssem)

        gen_block(ct, xh, blk)
        pltpu.async_copy(blk, out_hbm.at[h, ct, pl.ds(bt0, 8)], ssem)

    def pair_work(p, xh):
        def cp_body(cp, carry):
            phase(p, cp, 0, xh, blk0, ssem0)
            phase(p, cp, 1, xh, blk1, ssem1)
            return carry

        lax.fori_loop(0, _CT // 2, cp_body, 0)

    xh_start(0, xh0, isem0)

    def p_body(p, carry):
        @pl.when(lax.rem(p, 2) == 0)
        def _():
            xh_wait(xh0, isem0)
            xh_start(p + 1, xh1, isem1)
            pair_work(p, xh0)

        @pl.when(lax.rem(p, 2) == 1)
        def _():
            xh_wait(xh1, isem1)
            xh_start(p + 1, xh0, isem0)
            pair_work(p, xh1)

        return carry

    lax.fori_loop(0, _P_PW, p_body, 0)

    store_wait(blk0, ssem0)
    store_wait(blk1, ssem1)
    xh_wait(xh0, isem0)


@functools.partial(
    pl.kernel,
    out_type=jax.ShapeDtypeStruct((_HIST, _CT, _BT, 8, 128), jnp.float32),
    mesh=plsc.VectorSubcoreMesh(core_axis_name="c", subcore_axis_name="s"),
    scratch_types=[
        pltpu.VMEM((_D * 128,), jnp.float32),
        pltpu.VMEM((1024,), jnp.int32),
        pltpu.VMEM((1024,), jnp.int32),
        pltpu.VMEM((8, 8, 128), jnp.float32),
        pltpu.VMEM((8, 8, 128), jnp.float32),
        pltpu.SemaphoreType.DMA,
        pltpu.SemaphoreType.DMA,
        pltpu.SemaphoreType.DMA,
        pltpu.SemaphoreType.DMA,
    ],
    compiler_params=pltpu.CompilerParams(
        use_tc_tiling_on_sc=False, needs_layout_passes=False
    ),
)
def _sc_gather(tab_hbm, xt_hbm, out_hbm, tab_v, xh0, xh1, blk0, blk1, *sems):
    _gather_body(tab_hbm, xt_hbm, out_hbm, tab_v, xh0, xh1, blk0, blk1, sems)


def kernel(x, table):
    tab_t = _normalize_table_t(table).reshape(_D * 128)
    out5 = _sc_gather(tab_t, x.T)
    return out5.transpose(2, 4, 0, 1, 3).reshape(_BATCH, _HIST, _D)


# speedup vs baseline: 1.1387x; 1.0036x over previous
import functools

import jax
import jax.numpy as jnp
from jax import lax
from jax.experimental import pallas as pl
from jax.experimental.pallas import tpu as pltpu
from jax.experimental.pallas import tpu_sc as plsc

_IN_DIM = 101
_D = 64
_BATCH = 16384
_HIST = 200
_MAX_NORM = 1.0

_NC = 2
_NS = 16
_NW = _NC * _NS
_L = 16

_BT = _BATCH // 128
_CT = _D // 8
_BG = _BT // 8
_PAIRS = _HIST * _BG
_P_PW = _PAIRS // _NW

def _norm_body(tab_ref, out_ref):
    t = tab_ref[...]
    norms = jnp.sqrt(jnp.sum(t * t, axis=1, keepdims=True))
    scale = jnp.where(norms > _MAX_NORM, _MAX_NORM / (norms + 1e-7), 1.0)
    s = t * scale
    out_ref[...] = jnp.concatenate(
        [jnp.transpose(s), jnp.zeros((_D, 128 - _IN_DIM), jnp.float32)], axis=1
    )


def _normalize_table_t(table):
    return pl.pallas_call(
        _norm_body,
        out_shape=jax.ShapeDtypeStruct((_D, 128), jnp.float32),
    )(table)


def _gather_body(tab_hbm, xt_hbm, out_hbm, tab_v, xh0, xh1, blk0, blk1, sems):
    isem0, isem1, ssem0, ssem1 = sems
    pltpu.sync_copy(tab_hbm, tab_v)

    wid = lax.axis_index("s") * _NC + lax.axis_index("c")
    p_base = wid * _P_PW
    step128 = jnp.full((_L,), 128, jnp.int32)

    def xh_start(p, xh, isem):
        pp = lax.min(p_base + p, _PAIRS - 1)
        h = pp // _BG
        b0 = (pp % _BG) * 1024
        pltpu.async_copy(xt_hbm.at[h, pl.ds(b0, 1024)], xh, isem)

    def xh_wait(xh, isem):
        pltpu.make_async_copy(xt_hbm.at[0, pl.ds(0, 1024)], xh, isem).wait()

    def store_wait(blk, ssem):
        pltpu.make_async_copy(
            blk, out_hbm.at[0, 0, pl.ds(0, 8)], ssem
        ).wait()

    def gen_block(ct, xh, blk):
        base = jnp.full((_L,), ct * 1024, jnp.int32)

        @plsc.parallel_loop(0, 64, step=1, unroll=8)
        def _(g):
            bt = g // 8
            j16 = (g % 8) * _L
            xv = xh[pl.ds(g * _L, _L)]
            addr = xv + base
            for c8 in range(8):
                blk[bt, c8, pl.ds(j16, _L)] = plsc.load_gather(tab_v, [addr])
                if c8 != 7:
                    addr = addr + step128

    def phase(p, cp, ph, xh, blk, ssem):
        ct = cp * 2 + ph
        pp = p_base + p
        h = pp // _BG
        bt0 = (pp % _BG) * 8

        @pl.when(jnp.logical_or(p > 0, cp > 0))
        def _():
            store_wait(blk, ssem)

        gen_block(ct, xh, blk)
        pltpu.async_copy(blk, out_hbm.at[h, ct, pl.ds(bt0, 8)], ssem)

    def pair_work(p, xh):
        def cp_body(cp, carry):
            phase(p, cp, 0, xh, blk0, ssem0)
            phase(p, cp, 1, xh, blk1, ssem1)
            return carry

        lax.fori_loop(0, _CT // 2, cp_body, 0)

    xh_start(0, xh0, isem0)

    def p_body(p, carry):
        @pl.when(lax.rem(p, 2) == 0)
        def _():
            xh_wait(xh0, isem0)
            xh_start(p + 1, xh1, isem1)
            pair_work(p, xh0)

        @pl.when(lax.rem(p, 2) == 1)
        def _():
            xh_wait(xh1, isem1)
            xh_start(p + 1, xh0, isem0)
            pair_work(p, xh1)

        return carry

    lax.fori_loop(0, _P_PW, p_body, 0)

    store_wait(blk0, ssem0)
    store_wait(blk1, ssem1)
    xh_wait(xh0, isem0)


@functools.partial(
    pl.kernel,
    out_type=jax.ShapeDtypeStruct((_HIST, _CT, _BT, 8, 128), jnp.float32),
    mesh=plsc.VectorSubcoreMesh(core_axis_name="c", subcore_axis_name="s"),
    scratch_types=[
        pltpu.VMEM((_D * 128,), jnp.float32),
        pltpu.VMEM((1024,), jnp.int32),
        pltpu.VMEM((1024,), jnp.int32),
        pltpu.VMEM((8, 8, 128), jnp.float32),
        pltpu.VMEM((8, 8, 128), jnp.float32),
        pltpu.SemaphoreType.DMA,
        pltpu.SemaphoreType.DMA,
        pltpu.SemaphoreType.DMA,
        pltpu.SemaphoreType.DMA,
    ],
    compiler_params=pltpu.CompilerParams(
        use_tc_tiling_on_sc=False, needs_layout_passes=False
    ),
)
def _sc_gather(tab_hbm, xt_hbm, out_hbm, tab_v, xh0, xh1, blk0, blk1, *sems):
    _gather_body(tab_hbm, xt_hbm, out_hbm, tab_v, xh0, xh1, blk0, blk1, sems)


def kernel(x, table):
    tab_t = _normalize_table_t(table).reshape(_D * 128)
    out5 = _sc_gather(tab_t, x.T)
    return out5.transpose(2, 4, 0, 1, 3).reshape(_BATCH, _HIST, _D)
